# Initial kernel scaffold; baseline (speedup 1.0000x reference)
import jax
import jax.numpy as jnp
from jax.experimental import pallas as pl


def kernel(x, W):
    raise NotImplementedError("write your pallas kernel here")


# speedup vs baseline: 2.0668x; 2.0668x over previous
import functools

import jax
import jax.numpy as jnp
from jax.experimental import pallas as pl

D_MODEL = 768
NUM_EXPERTS = 64
TOP_K = 2


def _router_kernel(x_ref, w_ref, idx_ref, wt_ref):
    logits = jax.lax.dot_general(
        x_ref[...], w_ref[...],
        dimension_numbers=(((1,), (1,)), ((), ())),
        preferred_element_type=jnp.float32,
    )
    lane = jax.lax.broadcasted_iota(jnp.int32, logits.shape, 1)
    m1 = jnp.max(logits, axis=1, keepdims=True)
    i1 = jnp.min(jnp.where(logits == m1, lane, NUM_EXPERTS), axis=1, keepdims=True)
    masked = jnp.where(lane == i1, -jnp.inf, logits)
    m2 = jnp.max(masked, axis=1, keepdims=True)
    i2 = jnp.min(jnp.where(masked == m2, lane, NUM_EXPERTS), axis=1, keepdims=True)
    w1 = 1.0 / (1.0 + jnp.exp(m2 - m1))
    w2 = 1.0 - w1
    idx_ref[...] = jnp.concatenate([i1, i2], axis=1)
    wt_ref[...] = jnp.concatenate([w1, w2], axis=1)


@functools.partial(jax.jit, static_argnames=("block_tokens",))
def _route(x2d, W, block_tokens):
    n_tokens = x2d.shape[0]
    grid = (n_tokens // block_tokens,)
    idx, wts = pl.pallas_call(
        _router_kernel,
        grid=grid,
        in_specs=[
            pl.BlockSpec((block_tokens, D_MODEL), lambda i: (i, 0)),
            pl.BlockSpec((NUM_EXPERTS, D_MODEL), lambda i: (0, 0)),
        ],
        out_specs=[
            pl.BlockSpec((block_tokens, TOP_K), lambda i: (i, 0)),
            pl.BlockSpec((block_tokens, TOP_K), lambda i: (i, 0)),
        ],
        out_shape=[
            jax.ShapeDtypeStruct((n_tokens, TOP_K), jnp.int32),
            jax.ShapeDtypeStruct((n_tokens, TOP_K), jnp.float32),
        ],
    )(x2d, W)
    return idx, wts


def kernel(x, W):
    b, s, d = x.shape
    x2d = x.reshape(b * s, d)
    idx, wts = _route(x2d, W, 1024)
    return idx.reshape(b, s, TOP_K), wts.reshape(b, s, TOP_K)


# speedup vs baseline: 2.0671x; 1.0001x over previous
import functools

import jax
import jax.numpy as jnp
from jax.experimental import pallas as pl
from jax.experimental.pallas import tpu as pltpu

D_MODEL = 768
NUM_EXPERTS = 64
TOP_K = 2


def _router_kernel(x_ref, w_ref, idx_ref, wt_ref):
    logits = jax.lax.dot_general(
        x_ref[...], w_ref[...],
        dimension_numbers=(((1,), (1,)), ((), ())),
        preferred_element_type=jnp.float32,
    )
    lane = jax.lax.broadcasted_iota(jnp.int32, logits.shape, 1)
    m1 = jnp.max(logits, axis=1, keepdims=True)
    i1 = jnp.min(jnp.where(logits == m1, lane, NUM_EXPERTS), axis=1, keepdims=True)
    masked = jnp.where(lane == i1, -jnp.inf, logits)
    m2 = jnp.max(masked, axis=1, keepdims=True)
    i2 = jnp.min(jnp.where(masked == m2, lane, NUM_EXPERTS), axis=1, keepdims=True)
    w1 = 1.0 / (1.0 + jnp.exp(m2 - m1))
    w2 = 1.0 - w1
    idx_ref[...] = jnp.concatenate([i1, i2], axis=1)
    wt_ref[...] = jnp.concatenate([w1, w2], axis=1)


@functools.partial(jax.jit, static_argnames=("block_tokens",))
def _route(x2d, W, block_tokens):
    n_tokens = x2d.shape[0]
    grid = (n_tokens // block_tokens,)
    idx, wts = pl.pallas_call(
        _router_kernel,
        grid=grid,
        in_specs=[
            pl.BlockSpec((block_tokens, D_MODEL), lambda i: (i, 0)),
            pl.BlockSpec((NUM_EXPERTS, D_MODEL), lambda i: (0, 0)),
        ],
        out_specs=[
            pl.BlockSpec((block_tokens, TOP_K), lambda i: (i, 0)),
            pl.BlockSpec((block_tokens, TOP_K), lambda i: (i, 0)),
        ],
        out_shape=[
            jax.ShapeDtypeStruct((n_tokens, TOP_K), jnp.int32),
            jax.ShapeDtypeStruct((n_tokens, TOP_K), jnp.float32),
        ],
        compiler_params=pltpu.CompilerParams(
            dimension_semantics=("parallel",),
        ),
    )(x2d, W)
    return idx, wts


def kernel(x, W):
    b, s, d = x.shape
    x2d = x.reshape(b * s, d)
    idx, wts = _route(x2d, W, 1024)
    return idx.reshape(b, s, TOP_K), wts.reshape(b, s, TOP_K)


# speedup vs baseline: 2.3672x; 1.1452x over previous
import functools

import jax
import jax.numpy as jnp
from jax.experimental import pallas as pl
from jax.experimental.pallas import tpu as pltpu

D_MODEL = 768
NUM_EXPERTS = 64
TOP_K = 2


def _router_kernel(x_ref, w_ref, idx_ref, wt_ref):
    logits = jax.lax.dot_general(
        x_ref[...], w_ref[...],
        dimension_numbers=(((1,), (1,)), ((), ())),
        preferred_element_type=jnp.float32,
    )
    lane = jax.lax.broadcasted_iota(jnp.int32, logits.shape, 1)
    m1 = jnp.max(logits, axis=1, keepdims=True)
    i1 = jnp.min(jnp.where(logits == m1, lane, NUM_EXPERTS), axis=1, keepdims=True)
    masked = jnp.where(lane == i1, -jnp.inf, logits)
    m2 = jnp.max(masked, axis=1, keepdims=True)
    i2 = jnp.min(jnp.where(masked == m2, lane, NUM_EXPERTS), axis=1, keepdims=True)
    w1 = 1.0 / (1.0 + jnp.exp(m2 - m1))
    w2 = 1.0 - w1
    idx_ref[...] = jnp.concatenate([i1, i2], axis=1)
    wt_ref[...] = jnp.concatenate([w1, w2], axis=1)


@functools.partial(jax.jit, static_argnames=("block_tokens",))
def _route(x2d, W, block_tokens):
    n_tokens = x2d.shape[0]
    grid = (n_tokens // block_tokens,)
    idx, wts = pl.pallas_call(
        _router_kernel,
        grid=grid,
        in_specs=[
            pl.BlockSpec((block_tokens, D_MODEL), lambda i: (i, 0)),
            pl.BlockSpec((NUM_EXPERTS, D_MODEL), lambda i: (0, 0)),
        ],
        out_specs=[
            pl.BlockSpec((block_tokens, TOP_K), lambda i: (i, 0)),
            pl.BlockSpec((block_tokens, TOP_K), lambda i: (i, 0)),
        ],
        out_shape=[
            jax.ShapeDtypeStruct((n_tokens, TOP_K), jnp.int32),
            jax.ShapeDtypeStruct((n_tokens, TOP_K), jnp.float32),
        ],
        compiler_params=pltpu.CompilerParams(
            dimension_semantics=("parallel",),
        ),
    )(x2d, W)
    return idx, wts


def kernel(x, W):
    b, s, d = x.shape
    x2d = x.reshape(b * s, d)
    idx, wts = _route(x2d, W, 2048)
    return idx.reshape(b, s, TOP_K), wts.reshape(b, s, TOP_K)


# speedup vs baseline: 2.5492x; 1.0769x over previous
import functools

import jax
import jax.numpy as jnp
from jax.experimental import pallas as pl
from jax.experimental.pallas import tpu as pltpu

D_MODEL = 768
NUM_EXPERTS = 64
TOP_K = 2


def _router_kernel(x_ref, w_ref, idx_ref, wt_ref):
    logits = jax.lax.dot_general(
        x_ref[...], w_ref[...],
        dimension_numbers=(((1,), (1,)), ((), ())),
        preferred_element_type=jnp.float32,
    )
    lane = jax.lax.broadcasted_iota(jnp.int32, logits.shape, 1)
    m1 = jnp.max(logits, axis=1, keepdims=True)
    i1 = jnp.min(jnp.where(logits == m1, lane, NUM_EXPERTS), axis=1, keepdims=True)
    masked = jnp.where(lane == i1, -jnp.inf, logits)
    m2 = jnp.max(masked, axis=1, keepdims=True)
    i2 = jnp.min(jnp.where(masked == m2, lane, NUM_EXPERTS), axis=1, keepdims=True)
    w1 = 1.0 / (1.0 + jnp.exp(m2 - m1))
    w2 = 1.0 - w1
    idx_ref[...] = jnp.concatenate([i1, i2], axis=1)
    wt_ref[...] = jnp.concatenate([w1, w2], axis=1)


@functools.partial(jax.jit, static_argnames=("block_tokens",))
def _route(x2d, W, block_tokens):
    n_tokens = x2d.shape[0]
    grid = (n_tokens // block_tokens,)
    idx, wts = pl.pallas_call(
        _router_kernel,
        grid=grid,
        in_specs=[
            pl.BlockSpec((block_tokens, D_MODEL), lambda i: (i, 0)),
            pl.BlockSpec((NUM_EXPERTS, D_MODEL), lambda i: (0, 0)),
        ],
        out_specs=[
            pl.BlockSpec((block_tokens, TOP_K), lambda i: (i, 0)),
            pl.BlockSpec((block_tokens, TOP_K), lambda i: (i, 0)),
        ],
        out_shape=[
            jax.ShapeDtypeStruct((n_tokens, TOP_K), jnp.int32),
            jax.ShapeDtypeStruct((n_tokens, TOP_K), jnp.float32),
        ],
        compiler_params=pltpu.CompilerParams(
            dimension_semantics=("parallel",),
        ),
    )(x2d, W)
    return idx, wts


def kernel(x, W):
    b, s, d = x.shape
    x2d = x.reshape(b * s, d)
    idx, wts = _route(x2d, W, 4096)
    return idx.reshape(b, s, TOP_K), wts.reshape(b, s, TOP_K)


# speedup vs baseline: 4.5716x; 1.7933x over previous
import functools

import jax
import jax.numpy as jnp
from jax.experimental import pallas as pl
from jax.experimental.pallas import tpu as pltpu

D_MODEL = 768
NUM_EXPERTS = 64
TOP_K = 2


def _router_kernel(x_ref, w_ref, idx_ref, wt_ref):
    logits = jax.lax.dot_general(
        w_ref[...], x_ref[...],
        dimension_numbers=(((1,), (1,)), ((), ())),
        preferred_element_type=jnp.float32,
    )
    row = jax.lax.broadcasted_iota(jnp.int32, logits.shape, 0)
    m1 = jnp.max(logits, axis=0, keepdims=True)
    i1 = jnp.min(jnp.where(logits == m1, row, NUM_EXPERTS), axis=0, keepdims=True)
    masked = jnp.where(row == i1, -jnp.inf, logits)
    m2 = jnp.max(masked, axis=0, keepdims=True)
    i2 = jnp.min(jnp.where(masked == m2, row, NUM_EXPERTS), axis=0, keepdims=True)
    w1 = 1.0 / (1.0 + jnp.exp(m2 - m1))
    w2 = 1.0 - w1
    idx_ref[...] = jnp.concatenate([i1, i2], axis=0)
    wt_ref[...] = jnp.concatenate([w1, w2], axis=0)


@functools.partial(jax.jit, static_argnames=("block_tokens",))
def _route(x2d, W, block_tokens):
    n_tokens = x2d.shape[0]
    grid = (n_tokens // block_tokens,)
    idx_t, wts_t = pl.pallas_call(
        _router_kernel,
        grid=grid,
        in_specs=[
            pl.BlockSpec((block_tokens, D_MODEL), lambda i: (i, 0)),
            pl.BlockSpec((NUM_EXPERTS, D_MODEL), lambda i: (0, 0)),
        ],
        out_specs=[
            pl.BlockSpec((TOP_K, block_tokens), lambda i: (0, i)),
            pl.BlockSpec((TOP_K, block_tokens), lambda i: (0, i)),
        ],
        out_shape=[
            jax.ShapeDtypeStruct((TOP_K, n_tokens), jnp.int32),
            jax.ShapeDtypeStruct((TOP_K, n_tokens), jnp.float32),
        ],
        compiler_params=pltpu.CompilerParams(
            dimension_semantics=("parallel",),
        ),
    )(x2d, W)
    return idx_t, wts_t


def kernel(x, W):
    b, s, d = x.shape
    x2d = x.reshape(b * s, d)
    idx_t, wts_t = _route(x2d, W, 8192)
    idx = idx_t.T.reshape(b, s, TOP_K)
    wts = wts_t.T.reshape(b, s, TOP_K)
    return idx, wts


# speedup vs baseline: 4.9223x; 1.0767x over previous
import functools

import jax
import jax.numpy as jnp
from jax.experimental import pallas as pl
from jax.experimental.pallas import tpu as pltpu

D_MODEL = 768
NUM_EXPERTS = 64
TOP_K = 2


def _router_kernel(x_ref, w_ref, idx_ref, wt_ref):
    logits = jax.lax.dot_general(
        w_ref[...], x_ref[...],
        dimension_numbers=(((1,), (1,)), ((), ())),
        preferred_element_type=jnp.float32,
    )
    row = jax.lax.broadcasted_iota(jnp.int32, logits.shape, 0)
    m1 = jnp.max(logits, axis=0, keepdims=True)
    i1 = jnp.min(jnp.where(logits == m1, row, NUM_EXPERTS), axis=0, keepdims=True)
    masked = jnp.where(row == i1, -jnp.inf, logits)
    m2 = jnp.max(masked, axis=0, keepdims=True)
    i2 = jnp.min(jnp.where(masked == m2, row, NUM_EXPERTS), axis=0, keepdims=True)
    w1 = 1.0 / (1.0 + jnp.exp(m2 - m1))
    w2 = 1.0 - w1
    idx_ref[...] = jnp.concatenate([i1, i2], axis=0)
    wt_ref[...] = jnp.concatenate([w1, w2], axis=0)


@functools.partial(jax.jit, static_argnames=("block_tokens",))
def _route(x2d, W, block_tokens):
    n_tokens = x2d.shape[0]
    grid = (n_tokens // block_tokens,)
    idx_t, wts_t = pl.pallas_call(
        _router_kernel,
        grid=grid,
        in_specs=[
            pl.BlockSpec((block_tokens, D_MODEL), lambda i: (i, 0)),
            pl.BlockSpec((NUM_EXPERTS, D_MODEL), lambda i: (0, 0)),
        ],
        out_specs=[
            pl.BlockSpec((TOP_K, block_tokens), lambda i: (0, i)),
            pl.BlockSpec((TOP_K, block_tokens), lambda i: (0, i)),
        ],
        out_shape=[
            jax.ShapeDtypeStruct((TOP_K, n_tokens), jnp.int32),
            jax.ShapeDtypeStruct((TOP_K, n_tokens), jnp.float32),
        ],
        compiler_params=pltpu.CompilerParams(
            dimension_semantics=("parallel",),
        ),
    )(x2d, W)
    return idx_t, wts_t


def kernel(x, W):
    b, s, d = x.shape
    x2d = x.reshape(b * s, d)
    idx_t, wts_t = _route(x2d, W, 4096)
    idx = idx_t.T.reshape(b, s, TOP_K)
    wts = wts_t.T.reshape(b, s, TOP_K)
    return idx, wts
